# baseline (speedup 1.0000x reference)
import functools

import jax
import jax.numpy as jnp
from jax import lax
from jax.experimental import pallas as pl
from jax.experimental.pallas import tpu as pltpu
from jax.experimental.pallas import tpu_sc as plsc

_B = 16384
_N = 512
_L = 16
_NC = 2
_NS = 16
_NW = _NC * _NS
_RPW = _B // _NW
_R = 32
_NCHUNK = _RPW // _R
_G = _N // _L
_NIB = 4
_NOB = 2


@functools.partial(
    pl.kernel,
    mesh=plsc.VectorSubcoreMesh(core_axis_name="c", subcore_axis_name="s"),
    out_type=jax.ShapeDtypeStruct((_B, _N), jnp.float32),
    scratch_types=[
        pltpu.VMEM((_N,), jnp.int32),
        pltpu.VMEM((_N,), jnp.float32),
        pltpu.VMEM((_R, _N), jnp.float32),
        pltpu.VMEM((_R, _N), jnp.float32),
        pltpu.VMEM((_R, _N), jnp.float32),
        pltpu.VMEM((_R, _N), jnp.float32),
        pltpu.VMEM((_R, _N), jnp.float32),
        pltpu.VMEM((_R, _N), jnp.float32),
        pltpu.SemaphoreType.DMA,
        pltpu.SemaphoreType.DMA,
        pltpu.SemaphoreType.DMA,
        pltpu.SemaphoreType.DMA,
        pltpu.SemaphoreType.DMA,
        pltpu.SemaphoreType.DMA,
    ],
    compiler_params=pltpu.CompilerParams(needs_layout_passes=False),
)
def _gather_scale(x_hbm, w_hbm, ord_hbm, out_hbm,
                  ord_v, w_v, in0, in1, in2, in3, ou0, ou1,
                  si0, si1, si2, si3, so0, so1):
    wid = lax.axis_index("s") * _NC + lax.axis_index("c")
    row0 = wid * _RPW
    inb = (in0, in1, in2, in3)
    oub = (ou0, ou1)
    si = (si0, si1, si2, si3)
    so = (so0, so1)

    pltpu.sync_copy(ord_hbm, ord_v)
    pltpu.sync_copy(w_hbm, w_v)

    def start_in(c, b):
        pltpu.make_async_copy(
            x_hbm.at[pl.ds(row0 + c * _R, _R), :], inb[b], si[b]).start()

    def start_out(c, b):
        pltpu.make_async_copy(
            oub[b], out_hbm.at[pl.ds(row0 + c * _R, _R), :], so[b]).start()

    def wait_in(b):
        pltpu.make_async_copy(
            x_hbm.at[pl.ds(row0, _R), :], inb[b], si[b]).wait()

    def wait_out(b):
        pltpu.make_async_copy(
            oub[b], out_hbm.at[pl.ds(row0, _R), :], so[b]).wait()

    def check_body(j, ok):
        jm = j * _L
        idx = ord_v[pl.ds(jm, _L)]
        return jnp.logical_and(
            ok, jnp.all(idx == jm + lax.iota(jnp.int32, _L)))

    is_ident = lax.fori_loop(0, _G, check_body, True)

    def compute_fast(src, dst):
        for j in range(_G):
            jm = j * _L
            w = w_v[pl.ds(jm, _L)]
            for r in range(_R):
                dst[r, pl.ds(jm, _L)] = src[r, pl.ds(jm, _L)] * w

    def compute_slow(src, dst):
        def j_body(j, carry):
            jm = j * _L
            idx = ord_v[pl.ds(jm, _L)]
            w = w_v[pl.ds(jm, _L)]
            for r in range(_R):
                rv = jnp.full((_L,), r, jnp.int32)
                v = plsc.load_gather(src, [rv, idx])
                dst[r, pl.ds(jm, _L)] = v * w
            return carry
        lax.fori_loop(0, _G, j_body, 0)

    def compute(src, dst):
        @pl.when(is_ident)
        def _():
            compute_fast(src, dst)

        @pl.when(jnp.logical_not(is_ident))
        def _():
            compute_slow(src, dst)

    start_in(0, 0)
    start_in(1, 1)
    start_in(2, 2)

    def quad_body(cp, carry):
        for b in range(_NIB):
            c = _NIB * cp + b
            ob = b % _NOB
            wait_in(b)

            if b < _NOB:
                @pl.when(cp > 0)
                def _():
                    wait_out(ob)
            else:
                wait_out(ob)

            compute(inb[b], oub[ob])
            start_out(c, ob)

            if b == 0:
                start_in(c + _NIB - 1, (b + _NIB - 1) % _NIB)
            else:
                @pl.when(cp < _NCHUNK // _NIB - 1)
                def _():
                    start_in(c + _NIB - 1, (b + _NIB - 1) % _NIB)
        return carry

    lax.fori_loop(0, _NCHUNK // _NIB, quad_body, 0)
    wait_out(0)
    wait_out(1)


def kernel(x, weights, node_order):
    return _gather_scale(x, weights, node_order)


# speedup vs baseline: 1.1153x; 1.1153x over previous
import functools

import jax
import jax.numpy as jnp
from jax import lax
from jax.experimental import pallas as pl
from jax.experimental.pallas import tpu as pltpu
from jax.experimental.pallas import tpu_sc as plsc

_B = 16384
_N = 512
_L = 16
_NC = 2
_NS = 16
_NW = _NC * _NS
_RPW = _B // _NW
_R = 32
_NCHUNK = _RPW // _R
_G = _N // _L


@functools.partial(
    pl.kernel,
    mesh=plsc.VectorSubcoreMesh(core_axis_name="c", subcore_axis_name="s"),
    out_type=jax.ShapeDtypeStruct((_B, _N), jnp.float32),
    scratch_types=[
        pltpu.VMEM((_N,), jnp.int32),
        pltpu.VMEM((_N,), jnp.float32),
        pltpu.VMEM((_R, _N), jnp.float32),
        pltpu.VMEM((_R, _N), jnp.float32),
        pltpu.VMEM((_R, _N), jnp.float32),
        pltpu.VMEM((_R, _N), jnp.float32),
        pltpu.SemaphoreType.DMA,
        pltpu.SemaphoreType.DMA,
        pltpu.SemaphoreType.DMA,
        pltpu.SemaphoreType.DMA,
    ],
    compiler_params=pltpu.CompilerParams(
        needs_layout_passes=False, skip_device_barrier=True),
)
def _gather_scale(x_hbm, w_hbm, ord_hbm, out_hbm,
                  ord_v, w_v, in0, in1, ou0, ou1, si0, si1, so0, so1):
    wid = lax.axis_index("s") * _NC + lax.axis_index("c")
    row0 = wid * _RPW
    inb = (in0, in1)
    oub = (ou0, ou1)
    si = (si0, si1)
    so = (so0, so1)

    pltpu.sync_copy(ord_hbm, ord_v)
    pltpu.sync_copy(w_hbm, w_v)

    def start_in(c, b):
        pltpu.make_async_copy(
            x_hbm.at[pl.ds(row0 + c * _R, _R), :], inb[b], si[b]).start()

    def start_out(c, b):
        pltpu.make_async_copy(
            oub[b], out_hbm.at[pl.ds(row0 + c * _R, _R), :], so[b]).start()

    def wait_in(b):
        pltpu.make_async_copy(
            x_hbm.at[pl.ds(row0, _R), :], inb[b], si[b]).wait()

    def wait_out(b):
        pltpu.make_async_copy(
            oub[b], out_hbm.at[pl.ds(row0, _R), :], so[b]).wait()

    def check_body(j, ok):
        jm = j * _L
        idx = ord_v[pl.ds(jm, _L)]
        return jnp.logical_and(
            ok, jnp.all(idx == jm + lax.iota(jnp.int32, _L)))

    is_ident = lax.fori_loop(0, _G, check_body, True)

    def compute_fast(src, dst):
        for j in range(_G):
            jm = j * _L
            w = w_v[pl.ds(jm, _L)]
            for r in range(_R):
                dst[r, pl.ds(jm, _L)] = src[r, pl.ds(jm, _L)] * w

    def compute_slow(src, dst):
        def j_body(j, carry):
            jm = j * _L
            idx = ord_v[pl.ds(jm, _L)]
            w = w_v[pl.ds(jm, _L)]
            for r in range(_R):
                rv = jnp.full((_L,), r, jnp.int32)
                v = plsc.load_gather(src, [rv, idx])
                dst[r, pl.ds(jm, _L)] = v * w
            return carry
        lax.fori_loop(0, _G, j_body, 0)

    def compute(src, dst):
        @pl.when(is_ident)
        def _():
            compute_fast(src, dst)

        @pl.when(jnp.logical_not(is_ident))
        def _():
            compute_slow(src, dst)

    start_in(0, 0)
    start_in(1, 1)

    def pair_body(cp, carry):
        for b in (0, 1):
            c = 2 * cp + b
            wait_in(b)

            @pl.when(cp > 0)
            def _():
                wait_out(b)

            compute(inb[b], oub[b])
            start_out(c, b)

            @pl.when(cp < _NCHUNK // 2 - 1)
            def _():
                start_in(c + 2, b)
        return carry

    lax.fori_loop(0, _NCHUNK // 2, pair_body, 0)
    wait_out(0)
    wait_out(1)


def kernel(x, weights, node_order):
    return _gather_scale(x, weights, node_order)


# speedup vs baseline: 1.1388x; 1.0211x over previous
import functools

import jax
import jax.numpy as jnp
from jax import lax
from jax.experimental import pallas as pl
from jax.experimental.pallas import tpu as pltpu
from jax.experimental.pallas import tpu_sc as plsc

_B = 16384
_N = 512
_L = 16
_NC = 2
_NS = 16
_NW = _NC * _NS
_RPW = _B // _NW
_R = 32
_NCHUNK = _RPW // _R
_G = _N // _L


@functools.partial(
    pl.kernel,
    mesh=plsc.VectorSubcoreMesh(core_axis_name="c", subcore_axis_name="s"),
    out_type=jax.ShapeDtypeStruct((_B, _N), jnp.float32),
    scratch_types=[
        pltpu.VMEM((_N,), jnp.int32),
        pltpu.VMEM((_N,), jnp.float32),
        pltpu.VMEM((_R, _N), jnp.float32),
        pltpu.VMEM((_R, _N), jnp.float32),
        pltpu.VMEM((_R, _N), jnp.float32),
        pltpu.VMEM((_R, _N), jnp.float32),
        pltpu.SemaphoreType.DMA,
        pltpu.SemaphoreType.DMA,
        pltpu.SemaphoreType.DMA,
        pltpu.SemaphoreType.DMA,
    ],
    compiler_params=pltpu.CompilerParams(needs_layout_passes=False),
)
def _gather_scale(x_hbm, w_hbm, ord_hbm, out_hbm,
                  ord_v, w_v, in0, in1, ou0, ou1, si0, si1, so0, so1):
    wid = lax.axis_index("s") * _NC + lax.axis_index("c")
    row0 = wid * _RPW
    inb = (in0, in1)
    oub = (ou0, ou1)
    si = (si0, si1)
    so = (so0, so1)

    def start_in(c, b):
        pltpu.make_async_copy(
            x_hbm.at[pl.ds(row0 + c * _R, _R), :], inb[b], si[b]).start()

    def start_out(c, b):
        pltpu.make_async_copy(
            oub[b], out_hbm.at[pl.ds(row0 + c * _R, _R), :], so[b]).start()

    def wait_in(b):
        pltpu.make_async_copy(
            x_hbm.at[pl.ds(row0, _R), :], inb[b], si[b]).wait()

    def wait_out(b):
        pltpu.make_async_copy(
            oub[b], out_hbm.at[pl.ds(row0, _R), :], so[b]).wait()

    start_in(0, 0)
    start_in(1, 1)
    pltpu.sync_copy(ord_hbm, ord_v)
    pltpu.sync_copy(w_hbm, w_v)

    def check_body(j, ok):
        jm = j * _L
        idx = ord_v[pl.ds(jm, _L)]
        return jnp.logical_and(
            ok, jnp.all(idx == jm + lax.iota(jnp.int32, _L)))

    is_ident = lax.fori_loop(0, _G, check_body, True)

    def compute_fast(src, dst):
        for j in range(_G):
            jm = j * _L
            w = w_v[pl.ds(jm, _L)]
            for r in range(_R):
                dst[r, pl.ds(jm, _L)] = src[r, pl.ds(jm, _L)] * w

    def compute_slow(src, dst):
        def j_body(j, carry):
            jm = j * _L
            idx = ord_v[pl.ds(jm, _L)]
            w = w_v[pl.ds(jm, _L)]
            for r in range(_R):
                rv = jnp.full((_L,), r, jnp.int32)
                v = plsc.load_gather(src, [rv, idx])
                dst[r, pl.ds(jm, _L)] = v * w
            return carry
        lax.fori_loop(0, _G, j_body, 0)

    def compute(src, dst):
        @pl.when(is_ident)
        def _():
            compute_fast(src, dst)

        @pl.when(jnp.logical_not(is_ident))
        def _():
            compute_slow(src, dst)

    def pair_body(cp, carry):
        for b in (0, 1):
            c = 2 * cp + b
            wait_in(b)

            @pl.when(cp > 0)
            def _():
                wait_out(b)

            compute(inb[b], oub[b])
            start_out(c, b)

            @pl.when(cp < _NCHUNK // 2 - 1)
            def _():
                start_in(c + 2, b)
        return carry

    lax.fori_loop(0, _NCHUNK // 2, pair_body, 0)
    wait_out(0)
    wait_out(1)


def kernel(x, weights, node_order):
    return _gather_scale(x, weights, node_order)


# speedup vs baseline: 1.1809x; 1.0369x over previous
import functools

import jax
import jax.numpy as jnp
from jax import lax
from jax.experimental import pallas as pl
from jax.experimental.pallas import tpu as pltpu
from jax.experimental.pallas import tpu_sc as plsc

_B = 16384
_N = 512
_L = 16
_NC = 2
_NS = 16
_NW = _NC * _NS
_RPW = _B // _NW
_R = 16
_NCHUNK = _RPW // _R
_G = _N // _L


@functools.partial(
    pl.kernel,
    mesh=plsc.VectorSubcoreMesh(core_axis_name="c", subcore_axis_name="s"),
    out_type=jax.ShapeDtypeStruct((_B, _N), jnp.float32),
    scratch_types=[
        pltpu.VMEM((_N,), jnp.int32),
        pltpu.VMEM((_N,), jnp.float32),
        pltpu.VMEM((_R, _N), jnp.float32),
        pltpu.VMEM((_R, _N), jnp.float32),
        pltpu.VMEM((_R, _N), jnp.float32),
        pltpu.VMEM((_R, _N), jnp.float32),
        pltpu.SemaphoreType.DMA,
        pltpu.SemaphoreType.DMA,
        pltpu.SemaphoreType.DMA,
        pltpu.SemaphoreType.DMA,
    ],
    compiler_params=pltpu.CompilerParams(needs_layout_passes=False),
)
def _gather_scale(x_hbm, w_hbm, ord_hbm, out_hbm,
                  ord_v, w_v, in0, in1, ou0, ou1, si0, si1, so0, so1):
    wid = lax.axis_index("s") * _NC + lax.axis_index("c")
    row0 = wid * _RPW
    inb = (in0, in1)
    oub = (ou0, ou1)
    si = (si0, si1)
    so = (so0, so1)

    def start_in(c, b):
        pltpu.make_async_copy(
            x_hbm.at[pl.ds(row0 + c * _R, _R), :], inb[b], si[b]).start()

    def start_out(c, b):
        pltpu.make_async_copy(
            oub[b], out_hbm.at[pl.ds(row0 + c * _R, _R), :], so[b]).start()

    def wait_in(b):
        pltpu.make_async_copy(
            x_hbm.at[pl.ds(row0, _R), :], inb[b], si[b]).wait()

    def wait_out(b):
        pltpu.make_async_copy(
            oub[b], out_hbm.at[pl.ds(row0, _R), :], so[b]).wait()

    start_in(0, 0)
    start_in(1, 1)
    pltpu.sync_copy(ord_hbm, ord_v)
    pltpu.sync_copy(w_hbm, w_v)

    def check_body(j, ok):
        jm = j * _L
        idx = ord_v[pl.ds(jm, _L)]
        return jnp.logical_and(
            ok, jnp.all(idx == jm + lax.iota(jnp.int32, _L)))

    is_ident = lax.fori_loop(0, _G, check_body, True)

    def compute_fast(src, dst):
        for j in range(_G):
            jm = j * _L
            w = w_v[pl.ds(jm, _L)]
            for r in range(_R):
                dst[r, pl.ds(jm, _L)] = src[r, pl.ds(jm, _L)] * w

    def compute_slow(src, dst):
        def j_body(j, carry):
            jm = j * _L
            idx = ord_v[pl.ds(jm, _L)]
            w = w_v[pl.ds(jm, _L)]
            for r in range(_R):
                rv = jnp.full((_L,), r, jnp.int32)
                v = plsc.load_gather(src, [rv, idx])
                dst[r, pl.ds(jm, _L)] = v * w
            return carry
        lax.fori_loop(0, _G, j_body, 0)

    def compute(src, dst):
        @pl.when(is_ident)
        def _():
            compute_fast(src, dst)

        @pl.when(jnp.logical_not(is_ident))
        def _():
            compute_slow(src, dst)

    def pair_body(cp, carry):
        for b in (0, 1):
            c = 2 * cp + b
            wait_in(b)

            @pl.when(cp > 0)
            def _():
                wait_out(b)

            compute(inb[b], oub[b])
            start_out(c, b)

            @pl.when(cp < _NCHUNK // 2 - 1)
            def _():
                start_in(c + 2, b)
        return carry

    lax.fori_loop(0, _NCHUNK // 2, pair_body, 0)
    wait_out(0)
    wait_out(1)


def kernel(x, weights, node_order):
    return _gather_scale(x, weights, node_order)
